# baseline (speedup 1.0000x reference)
import functools

import jax
import jax.numpy as jnp
from jax import lax
from jax.experimental import pallas as pl
from jax.experimental.pallas import tpu as pltpu
from jax.experimental.pallas import tpu_sc as plsc

_V = 1000000
_D = 32
_B = 16384
_NC = 2
_NS = 16
_NW = _NC * _NS
_BPW = _B // _NW
_L = 16
_GRPS = _BPW // _L


def _make_sc_kernel():
    mesh = plsc.VectorSubcoreMesh(
        core_axis_name="c", subcore_axis_name="s",
        num_cores=_NC, num_subcores=_NS)

    @functools.partial(
        pl.kernel,
        out_type=jax.ShapeDtypeStruct((_B,), jnp.float32),
        mesh=mesh,
        compiler_params=pltpu.CompilerParams(
            needs_layout_passes=False,
            use_tc_tiling_on_sc=False,
        ),
        scratch_types=[
            pltpu.VMEM((_BPW,), jnp.int32),
            pltpu.VMEM((_BPW,), jnp.int32),
            pltpu.VMEM((_D, _BPW), jnp.float32),
            pltpu.VMEM((_D, _BPW), jnp.float32),
            pltpu.VMEM((_BPW,), jnp.float32),
            pltpu.VMEM((_BPW,), jnp.float32),
            pltpu.VMEM((_BPW,), jnp.float32),
            pltpu.SemaphoreType.DMA,
            pltpu.SemaphoreType.DMA,
        ],
    )
    def sc_kernel(ids1_hbm, ids2_hbm, w1t_hbm, b1_hbm, w2t_hbm, b2_hbm,
                  out_hbm, ids1_v, ids2_v, g1_v, g2_v,
                  bias1_v, bias2_v, out_v, sem_w, sem_b):
        wid = lax.axis_index("s") * _NC + lax.axis_index("c")
        base = wid * _BPW

        pltpu.sync_copy(ids1_hbm.at[pl.ds(base, _BPW)], ids1_v)
        pltpu.sync_copy(ids2_hbm.at[pl.ds(base, _BPW)], ids2_v)

        cb1 = pltpu.async_copy(b1_hbm.at[ids1_v], bias1_v, sem_b)
        cb2 = pltpu.async_copy(b2_hbm.at[ids2_v], bias2_v, sem_b)
        copies = []
        for d in range(_D):
            copies.append(pltpu.async_copy(
                w1t_hbm.at[d].at[ids1_v], g1_v.at[d], sem_w))
            copies.append(pltpu.async_copy(
                w2t_hbm.at[d].at[ids2_v], g2_v.at[d], sem_w))
        cb1.wait()
        cb2.wait()
        for c in copies:
            c.wait()

        def group(g, carry):
            j0 = g * _L
            acc = bias1_v[pl.ds(j0, _L)] + bias2_v[pl.ds(j0, _L)]
            for d in range(_D):
                acc = acc + g1_v[d, pl.ds(j0, _L)] * g2_v[d, pl.ds(j0, _L)]
            out_v[pl.ds(j0, _L)] = acc
            return carry

        lax.fori_loop(0, _GRPS, group, 0)

        pltpu.sync_copy(out_v, out_hbm.at[pl.ds(base, _BPW)])

    return sc_kernel


_SC_KERNEL = _make_sc_kernel()


def kernel(ids1, ids2, W1, b1, W2, b2):
    return _SC_KERNEL(ids1.astype(jnp.int32), ids2.astype(jnp.int32),
                      W1.T, b1, W2.T, b2)


# speedup vs baseline: 19.1149x; 19.1149x over previous
import functools

import jax
import jax.numpy as jnp
from jax import lax
from jax.experimental import pallas as pl
from jax.experimental.pallas import tpu as pltpu
from jax.experimental.pallas import tpu_sc as plsc

_V = 1000000
_D = 32
_B = 16384
_NC = 2
_NS = 16
_NW = _NC * _NS
_BPW = _B // _NW
_L = 16
_WAVE = 8


def _make_sc_kernel():
    mesh = plsc.VectorSubcoreMesh(
        core_axis_name="c", subcore_axis_name="s",
        num_cores=_NC, num_subcores=_NS)

    @functools.partial(
        pl.kernel,
        out_type=jax.ShapeDtypeStruct((_B,), jnp.float32),
        mesh=mesh,
        compiler_params=pltpu.CompilerParams(
            needs_layout_passes=False,
            disable_bounds_checks=True,
        ),
        scratch_types=[
            pltpu.VMEM((_BPW,), jnp.int32),
            pltpu.VMEM((_BPW,), jnp.int32),
            pltpu.VMEM((_WAVE * 64, 128), jnp.float32),
            pltpu.VMEM((_WAVE * 2, 128), jnp.float32),
            pltpu.VMEM((_BPW,), jnp.float32),
            pltpu.SemaphoreType.DMA,
        ],
    )
    def sc_kernel(ids1_hbm, ids2_hbm, w1_hbm, b1_hbm, w2_hbm, b2_hbm,
                  out_hbm, ids1_v, ids2_v, chunks_v, bchunks_v, out_v, sem):
        wid = lax.axis_index("s") * _NC + lax.axis_index("c")
        base = wid * _BPW

        pltpu.sync_copy(ids1_hbm.at[pl.ds(base, _BPW)], ids1_v)
        pltpu.sync_copy(ids2_hbm.at[pl.ds(base, _BPW)], ids2_v)

        iota = lax.iota(jnp.int32, _L)
        izeros = lax.broadcast(jnp.int32(0), (_L,))
        zeros = lax.broadcast(jnp.float32(0), (_L,))
        lane0 = iota == 0
        lane15 = iota == 15

        def block(k, carry):
            idv1 = ids1_v[pl.ds(k * _L, _L)]
            idv2 = ids2_v[pl.ds(k * _L, _L)]
            for half in range(2):
                j0 = k * _L + half * _WAVE
                scalars = []
                copies = []
                for s in range(_WAVE):
                    lane = half * _WAVE + s
                    i1 = jnp.sum(jnp.where(iota == lane, idv1, izeros))
                    i2 = jnp.sum(jnp.where(iota == lane, idv2, izeros))
                    scalars.append((i1, i2))
                    for tab, (w_hbm, b_hbm, i) in enumerate(
                            ((w1_hbm, b1_hbm, i1), (w2_hbm, b2_hbm, i2))):
                        col = pl.multiple_of((i // 128) * 128, 128)
                        for p in range(4):
                            copies.append(pltpu.async_copy(
                                w_hbm.at[p, :, pl.ds(col, 128)],
                                chunks_v.at[
                                    pl.ds(s * 64 + tab * 32 + p * 8, 8), :],
                                sem))
                        copies.append(pltpu.async_copy(
                            b_hbm.at[pl.ds(col, 128)],
                            bchunks_v.at[s * 2 + tab], sem))
                for c in copies:
                    c.wait()
                for s in range(_WAVE):
                    i1, i2 = scalars[s]
                    c1 = lax.broadcast(i1 % 128, (_L,))
                    c2 = lax.broadcast(i2 % 128, (_L,))
                    r0 = s * 64
                    w1lo = plsc.load_gather(chunks_v, [r0 + iota, c1])
                    w1hi = plsc.load_gather(chunks_v, [r0 + 16 + iota, c1])
                    w2lo = plsc.load_gather(chunks_v, [r0 + 32 + iota, c2])
                    w2hi = plsc.load_gather(chunks_v, [r0 + 48 + iota, c2])
                    b1v = plsc.load_gather(
                        bchunks_v,
                        [lax.broadcast(jnp.int32(s * 2), (_L,)), c1])
                    b2v = plsc.load_gather(
                        bchunks_v,
                        [lax.broadcast(jnp.int32(s * 2 + 1), (_L,)), c2])
                    sv = w1lo * w2lo + w1hi * w2hi
                    sv = sv + jnp.where(lane0, b1v + b2v, zeros)
                    cs = plsc.cumsum(sv)
                    plsc.store_scatter(out_v, [lax.broadcast(j0 + s, (_L,))],
                                       cs, mask=lane15)
            return carry

        lax.fori_loop(0, _BPW // _L, block, 0)

        pltpu.sync_copy(out_v, out_hbm.at[pl.ds(base, _BPW)])

    return sc_kernel


_SC_KERNEL = _make_sc_kernel()


def kernel(ids1, ids2, W1, b1, W2, b2):
    w1 = W1.T.reshape(4, 8, _V)
    w2 = W2.T.reshape(4, 8, _V)
    return _SC_KERNEL(ids1.astype(jnp.int32), ids2.astype(jnp.int32),
                      w1, b1, w2, b2)


# speedup vs baseline: 19.1931x; 1.0041x over previous
import functools

import jax
import jax.numpy as jnp
from jax import lax
from jax.experimental import pallas as pl
from jax.experimental.pallas import tpu as pltpu
from jax.experimental.pallas import tpu_sc as plsc

_V = 1000000
_D = 32
_B = 16384
_NC = 2
_NS = 16
_NW = _NC * _NS
_BPW = _B // _NW
_L = 16
_WAVE = 8


def _make_sc_kernel():
    mesh = plsc.VectorSubcoreMesh(
        core_axis_name="c", subcore_axis_name="s",
        num_cores=_NC, num_subcores=_NS)

    @functools.partial(
        pl.kernel,
        out_type=jax.ShapeDtypeStruct((_B,), jnp.float32),
        mesh=mesh,
        compiler_params=pltpu.CompilerParams(
            needs_layout_passes=False,
            disable_bounds_checks=True,
        ),
        scratch_types=[
            pltpu.VMEM((_BPW,), jnp.int32),
            pltpu.VMEM((_BPW,), jnp.int32),
            pltpu.VMEM((_WAVE * 64, 128), jnp.float32),
            pltpu.VMEM((_WAVE * 2, 128), jnp.float32),
            pltpu.VMEM((_BPW,), jnp.float32),
            pltpu.SemaphoreType.DMA,
        ],
    )
    def sc_kernel(ids1_hbm, ids2_hbm, w1_hbm, b1_hbm, w2_hbm, b2_hbm,
                  out_hbm, ids1_v, ids2_v, chunks_v, bchunks_v, out_v, sem):
        wid = lax.axis_index("s") * _NC + lax.axis_index("c")
        base = wid * _BPW

        pltpu.sync_copy(ids1_hbm.at[pl.ds(base, _BPW)], ids1_v)
        pltpu.sync_copy(ids2_hbm.at[pl.ds(base, _BPW)], ids2_v)

        iota = lax.iota(jnp.int32, _L)
        izeros = lax.broadcast(jnp.int32(0), (_L,))
        zeros = lax.broadcast(jnp.float32(0), (_L,))
        lane0 = iota == 0
        lane15 = iota == 15

        def block(k, carry):
            idv1 = ids1_v[pl.ds(k * _L, _L)]
            idv2 = ids2_v[pl.ds(k * _L, _L)]
            for half in range(2):
                j0 = k * _L + half * _WAVE
                scalars = []
                copies = []
                for s in range(_WAVE):
                    lane = half * _WAVE + s
                    i1 = jnp.sum(jnp.where(iota == lane, idv1, izeros))
                    i2 = jnp.sum(jnp.where(iota == lane, idv2, izeros))
                    scalars.append((i1, i2))
                    for tab, (w_hbm, b_hbm, i) in enumerate(
                            ((w1_hbm, b1_hbm, i1), (w2_hbm, b2_hbm, i2))):
                        col = pl.multiple_of((i // 128) * 128, 128)
                        copies.append(pltpu.async_copy(
                            w_hbm.at[:, pl.ds(col, 128)],
                            chunks_v.at[pl.ds(s * 64 + tab * 32, 32), :],
                            sem))
                        copies.append(pltpu.async_copy(
                            b_hbm.at[pl.ds(col, 128)],
                            bchunks_v.at[s * 2 + tab], sem))
                for c in copies:
                    c.wait()
                for s in range(_WAVE):
                    i1, i2 = scalars[s]
                    c1 = lax.broadcast(i1 % 128, (_L,))
                    c2 = lax.broadcast(i2 % 128, (_L,))
                    r0 = s * 64
                    w1lo = plsc.load_gather(chunks_v, [r0 + iota, c1])
                    w1hi = plsc.load_gather(chunks_v, [r0 + 16 + iota, c1])
                    w2lo = plsc.load_gather(chunks_v, [r0 + 32 + iota, c2])
                    w2hi = plsc.load_gather(chunks_v, [r0 + 48 + iota, c2])
                    b1v = plsc.load_gather(
                        bchunks_v,
                        [lax.broadcast(jnp.int32(s * 2), (_L,)), c1])
                    b2v = plsc.load_gather(
                        bchunks_v,
                        [lax.broadcast(jnp.int32(s * 2 + 1), (_L,)), c2])
                    sv = w1lo * w2lo + w1hi * w2hi
                    sv = sv + jnp.where(lane0, b1v + b2v, zeros)
                    cs = plsc.cumsum(sv)
                    plsc.store_scatter(out_v, [lax.broadcast(j0 + s, (_L,))],
                                       cs, mask=lane15)
            return carry

        lax.fori_loop(0, _BPW // _L, block, 0)

        pltpu.sync_copy(out_v, out_hbm.at[pl.ds(base, _BPW)])

    return sc_kernel


_SC_KERNEL = _make_sc_kernel()


def kernel(ids1, ids2, W1, b1, W2, b2):
    return _SC_KERNEL(ids1.astype(jnp.int32), ids2.astype(jnp.int32),
                      W1.T, b1, W2.T, b2)
